# baseline (speedup 1.0000x reference)
import dataclasses
import functools

import jax
import jax.numpy as jnp
from jax import lax
from jax.experimental import pallas as pl
from jax.experimental.pallas import tpu as pltpu
from jax.experimental.pallas import tpu_sc as plsc

N_WORKERS = 32
LANES = 16
D_FEAT = 128
CHUNK = 160
NBUF = 2


@functools.cache
def _edge_dot_fn(E: int):
    epw = E // N_WORKERS
    n_chunks = epw // CHUNK
    assert epw % CHUNK == 0 and CHUNK % LANES == 0 and epw % 8 == 0
    assert n_chunks % NBUF == 0 and n_chunks >= 3 * NBUF

    mesh = plsc.VectorSubcoreMesh(core_axis_name="c", subcore_axis_name="s")

    cp = pltpu.CompilerParams()
    if "needs_layout_passes" in pltpu.CompilerParams.__dataclass_fields__:
        cp = dataclasses.replace(cp, needs_layout_passes=False)

    @functools.partial(
        pl.kernel,
        compiler_params=cp,
        out_type=jax.ShapeDtypeStruct((E,), jnp.float32),
        mesh=mesh,
        scratch_types=[
            pltpu.VMEM((epw,), jnp.int32),
            pltpu.VMEM((epw,), jnp.int32),
            pltpu.VMEM((CHUNK, D_FEAT), jnp.float32),
            pltpu.VMEM((CHUNK, D_FEAT), jnp.float32),
            pltpu.VMEM((CHUNK, D_FEAT), jnp.float32),
            pltpu.VMEM((CHUNK, D_FEAT), jnp.float32),
            pltpu.VMEM((CHUNK,), jnp.float32),
            pltpu.VMEM((CHUNK,), jnp.float32),
            pltpu.VMEM((LANES, LANES), jnp.float32),
            pltpu.SemaphoreType.DMA,
            pltpu.SemaphoreType.DMA,
            pltpu.SemaphoreType.DMA,
            pltpu.SemaphoreType.DMA,
            pltpu.SemaphoreType.DMA,
            pltpu.SemaphoreType.DMA,
        ],
    )
    def kern(h_hbm, src_hbm, dst_hbm, out_hbm,
             sidx, didx, u0, u1, v0, v1, o0, o1, acc_v,
             su0, sv0, su1, sv1, so0, so1):
        wid = lax.axis_index("s") * 2 + lax.axis_index("c")
        base = wid * epw
        u_v = (u0, u1)
        v_v = (v0, v1)
        out_v = (o0, o1)
        sem_u = (su0, su1)
        sem_v = (sv0, sv1)
        sem_o = (so0, so1)

        pltpu.sync_copy(src_hbm.at[pl.ds(base, epw)], sidx)
        pltpu.sync_copy(dst_hbm.at[pl.ds(base, epw)], didx)

        def gathers(j, b):
            off = j * CHUNK
            return (
                pltpu.make_async_copy(
                    h_hbm.at[sidx.at[pl.ds(off, CHUNK)]], u_v[b], sem_u[b]),
                pltpu.make_async_copy(
                    h_hbm.at[didx.at[pl.ds(off, CHUNK)]], v_v[b], sem_v[b]),
            )

        def out_copy(j, b):
            return pltpu.make_async_copy(
                out_v[b], out_hbm.at[pl.ds(base + j * CHUNK, CHUNK)],
                sem_o[b])

        def start_gathers(j, b):
            for c in gathers(j, b):
                c.start()

        def wait_gathers(j, b):
            for c in gathers(j, b):
                c.wait()

        def compute(j, b):
            @pl.loop(0, CHUNK, step=LANES)
            def _(g):
                for e in range(LANES):
                    a = (u_v[b][g + e, pl.ds(0, LANES)]
                         * v_v[b][g + e, pl.ds(0, LANES)])
                    for s_ in range(1, D_FEAT // LANES):
                        a += (u_v[b][g + e, pl.ds(s_ * LANES, LANES)]
                              * v_v[b][g + e, pl.ds(s_ * LANES, LANES)])
                    acc_v[e] = a
                rows_i = lax.iota(jnp.int32, LANES)
                s_vec = jnp.zeros((LANES,), jnp.float32)
                for f in range(LANES):
                    cols_i = jnp.full((LANES,), f, jnp.int32)
                    s_vec += plsc.load_gather(acc_v, [rows_i, cols_i])
                out_v[b][pl.ds(g, LANES)] = s_vec

        for b in range(NBUF):
            start_gathers(b, b)
        for b in range(NBUF):
            wait_gathers(b, b)
            compute(b, b)
            out_copy(b, b).start()
            start_gathers(b + NBUF, b)

        @pl.loop(NBUF, n_chunks - NBUF, step=NBUF)
        def _(g):
            for b in range(NBUF):
                j = g + b
                wait_gathers(j, b)
                out_copy(j - NBUF, b).wait()
                compute(j, b)
                out_copy(j, b).start()
                start_gathers(j + NBUF, b)

        for b in range(NBUF):
            j = n_chunks - NBUF + b
            wait_gathers(j, b)
            out_copy(j - NBUF, b).wait()
            compute(j, b)
            out_copy(j, b).start()
        for b in range(NBUF):
            out_copy(n_chunks - NBUF + b, b).wait()

    return kern


def kernel(h, edge_index):
    E = edge_index.shape[1]
    step = N_WORKERS * CHUNK * NBUF
    E_pad = ((E + step - 1) // step) * step
    src = edge_index[0].astype(jnp.int32)
    dst = edge_index[1].astype(jnp.int32)
    if E_pad != E:
        src = jnp.pad(src, (0, E_pad - E))
        dst = jnp.pad(dst, (0, E_pad - E))
    out = _edge_dot_fn(E_pad)(h, src, dst)
    return out[:E] if E_pad != E else out


# speedup vs baseline: 2.6623x; 2.6623x over previous
import dataclasses
import functools

import jax
import jax.numpy as jnp
from jax import lax
from jax.experimental import pallas as pl
from jax.experimental.pallas import tpu as pltpu
from jax.experimental.pallas import tpu_sc as plsc

N_WORKERS = 32
LANES = 16
D_FEAT = 128
CHUNK = 400


@functools.cache
def _edge_dot_fn(E: int):
    epw = E // N_WORKERS
    n_chunks = epw // CHUNK
    assert epw % CHUNK == 0 and CHUNK % LANES == 0 and epw % 8 == 0

    mesh = plsc.VectorSubcoreMesh(core_axis_name="c", subcore_axis_name="s")

    cp = pltpu.CompilerParams()
    if "needs_layout_passes" in pltpu.CompilerParams.__dataclass_fields__:
        cp = dataclasses.replace(cp, needs_layout_passes=False)

    @functools.partial(
        pl.kernel,
        compiler_params=cp,
        out_type=jax.ShapeDtypeStruct((E,), jnp.float32),
        mesh=mesh,
        scratch_types=[
            pltpu.VMEM((epw,), jnp.int32),
            pltpu.VMEM((epw,), jnp.int32),
            pltpu.VMEM((CHUNK, D_FEAT), jnp.float32),
            pltpu.VMEM((CHUNK, D_FEAT), jnp.float32),
            pltpu.VMEM((CHUNK,), jnp.float32),
            pltpu.VMEM((LANES, LANES), jnp.float32),
            pltpu.SemaphoreType.DMA,
            pltpu.SemaphoreType.DMA,
        ],
    )
    def kern(h_hbm, src_hbm, dst_hbm, out_hbm,
             sidx, didx, u_v, v_v, out_v, acc_v, sem_u, sem_v):
        wid = lax.axis_index("s") * 2 + lax.axis_index("c")
        base = wid * epw

        pltpu.sync_copy(src_hbm.at[pl.ds(base, epw)], sidx)
        pltpu.sync_copy(dst_hbm.at[pl.ds(base, epw)], didx)

        @pl.loop(0, n_chunks)
        def _(j):
            off = j * CHUNK
            cu = pltpu.async_copy(
                h_hbm.at[sidx.at[pl.ds(off, CHUNK)]], u_v, sem_u)
            cv = pltpu.async_copy(
                h_hbm.at[didx.at[pl.ds(off, CHUNK)]], v_v, sem_v)
            cu.wait()
            cv.wait()

            @pl.loop(0, CHUNK, step=LANES)
            def _(g):
                for e in range(LANES):
                    a = (u_v[g + e, pl.ds(0, LANES)]
                         * v_v[g + e, pl.ds(0, LANES)])
                    for s_ in range(1, D_FEAT // LANES):
                        a += (u_v[g + e, pl.ds(s_ * LANES, LANES)]
                              * v_v[g + e, pl.ds(s_ * LANES, LANES)])
                    acc_v[e] = a
                rows_i = lax.iota(jnp.int32, LANES)
                s_vec = jnp.zeros((LANES,), jnp.float32)
                for f in range(LANES):
                    cols_i = jnp.full((LANES,), f, jnp.int32)
                    s_vec += plsc.load_gather(acc_v, [rows_i, cols_i])
                out_v[pl.ds(g, LANES)] = s_vec

            pltpu.sync_copy(out_v, out_hbm.at[pl.ds(base + off, CHUNK)])

    return kern


def kernel(h, edge_index):
    src = edge_index[0].astype(jnp.int32)
    dst = edge_index[1].astype(jnp.int32)
    return _edge_dot_fn(edge_index.shape[1])(h, src, dst)
